# baseline (speedup 1.0000x reference)
import jax
import jax.numpy as jnp
from jax import lax
from jax.experimental import pallas as pl
from jax.experimental.pallas import tpu as pltpu
from jax.experimental.pallas import tpu_sc as plsc

L_OUT = 384
D = 256
ROWS_PER_WORKER = 12
T_ROWS = 760
WIN_ROWS = 472


def _phase_body(table_ref, t_ref):
    for c in range(8):
        t_ref[c] = table_ref[pl.ds(124 - c, T_ROWS), :]


def _pairwise_body(t_hbm, out_hbm, win, sem):
    c_ax = lax.axis_index("c")
    s_ax = lax.axis_index("s")
    wid = s_ax * 2 + c_ax
    c = wid % 8
    m = wid // 8
    w0 = pl.multiple_of(288 - 96 * m, 8)
    pltpu.sync_copy(t_hbm.at[c, pl.ds(w0, WIN_ROWS)], win)
    copies = []
    for t in range(ROWS_PER_WORKER):
        for h in range(2):
            copies.append(
                pltpu.async_copy(
                    win.at[pl.ds(8 * (ROWS_PER_WORKER - 1 - t) + 192 * h,
                                 L_OUT // 2)],
                    out_hbm.at[c + 96 * m + 8 * t, pl.ds(192 * h, L_OUT // 2)],
                    sem,
                )
            )
    for cp in copies:
        cp.wait()


def kernel(L, rel_pos_embed):
    t = pl.pallas_call(
        _phase_body,
        out_shape=jax.ShapeDtypeStruct((8, T_ROWS, D), jnp.float32),
    )(rel_pos_embed)
    mesh = plsc.VectorSubcoreMesh(core_axis_name="c", subcore_axis_name="s")
    run = pl.kernel(
        _pairwise_body,
        out_type=jax.ShapeDtypeStruct((L_OUT, L_OUT, D), jnp.float32),
        mesh=mesh,
        scratch_types=[
            pltpu.VMEM((WIN_ROWS, D), jnp.float32),
            pltpu.SemaphoreType.DMA,
        ],
    )
    return run(t)


# speedup vs baseline: 1.0085x; 1.0085x over previous
import jax
import jax.numpy as jnp
from jax import lax
from jax.experimental import pallas as pl
from jax.experimental.pallas import tpu as pltpu
from jax.experimental.pallas import tpu_sc as plsc

L_OUT = 384
D = 256
ROWS_PER_WORKER = 12
T_ROWS = 760
WIN_ROWS = 472


def _phase_body(table_ref, t_ref):
    for c in range(8):
        t_ref[c] = table_ref[pl.ds(124 - c, T_ROWS), :]


def _pairwise_body(t_hbm, out_hbm, win, sem):
    c_ax = lax.axis_index("c")
    s_ax = lax.axis_index("s")
    wid = s_ax * 2 + c_ax
    c = wid % 8
    m = wid // 8
    w0 = pl.multiple_of(288 - 96 * m, 8)
    pltpu.sync_copy(t_hbm.at[c, pl.ds(w0, WIN_ROWS)], win)
    copies = []
    for t in range(ROWS_PER_WORKER):
        copies.append(
            pltpu.async_copy(
                win.at[pl.ds(8 * (ROWS_PER_WORKER - 1 - t), L_OUT)],
                out_hbm.at[c + 96 * m + 8 * t],
                sem,
            )
        )
    for cp in copies:
        cp.wait()


def kernel(L, rel_pos_embed):
    t = pl.pallas_call(
        _phase_body,
        out_shape=jax.ShapeDtypeStruct((8, T_ROWS, D), jnp.float32),
    )(rel_pos_embed)
    mesh = plsc.VectorSubcoreMesh(core_axis_name="c", subcore_axis_name="s")
    run = pl.kernel(
        _pairwise_body,
        out_type=jax.ShapeDtypeStruct((L_OUT, L_OUT, D), jnp.float32),
        mesh=mesh,
        scratch_types=[
            pltpu.VMEM((WIN_ROWS, D), jnp.float32),
            pltpu.SemaphoreType.DMA,
        ],
    )
    return run(t)
